# initial kernel scaffold (unmeasured)
import jax
import jax.numpy as jnp
from jax import lax
from jax.experimental import pallas as pl
from jax.experimental.pallas import tpu as pltpu

N_DEV = 4
S_PER = 1024
D_MODEL = 1024
HQ = 8
DH = 128
BLK = 64
SCALE = 0.08838834764831843
NEG = -1e9


def kernel(x, Wq, K_ext, V_ext, Wo):
    x2 = x.reshape(S_PER, D_MODEL)
    k2 = K_ext.reshape(S_PER, HQ * DH)
    v2 = V_ext.reshape(S_PER, HQ * DH)

    def body(x_ref, wq_ref, k_ref, v_ref, wo_ref, out_ref,
             k_all, v_all, ksend, krecv, vsend, vrecv):
        my = lax.axis_index("i")
        left = lax.rem(my + (N_DEV - 1), N_DEV)
        right = lax.rem(my + 1, N_DEV)

        barrier_sem = pltpu.get_barrier_semaphore()
        for nbr in (left, right):
            pl.semaphore_signal(
                barrier_sem, inc=1,
                device_id=(nbr,), device_id_type=pl.DeviceIdType.MESH,
            )
        pl.semaphore_wait(barrier_sem, 2)

        k_all[0] = k_ref[...].astype(jnp.bfloat16)
        v_all[0] = v_ref[...].astype(jnp.bfloat16)

        for h in range(1, N_DEV):
            kr = pltpu.make_async_remote_copy(
                src_ref=k_all.at[h - 1], dst_ref=k_all.at[h],
                send_sem=ksend.at[h - 1], recv_sem=krecv.at[h - 1],
                device_id=(right,), device_id_type=pl.DeviceIdType.MESH,
            )
            vr = pltpu.make_async_remote_copy(
                src_ref=v_all.at[h - 1], dst_ref=v_all.at[h],
                send_sem=vsend.at[h - 1], recv_sem=vrecv.at[h - 1],
                device_id=(right,), device_id_type=pl.DeviceIdType.MESH,
            )
            kr.start()
            vr.start()
            kr.wait()
            vr.wait()

        q = lax.dot_general(
            x2_cast := x_ref[...].astype(jnp.bfloat16),
            wq_ref[...].astype(jnp.bfloat16),
            (((1,), (0,)), ((), ())),
            preferred_element_type=jnp.float32,
        ).astype(jnp.bfloat16)

        r = lax.broadcasted_iota(jnp.int32, (S_PER, S_PER), 0)
        c = lax.broadcasted_iota(jnp.int32, (S_PER, S_PER), 1)
        local_bias = jnp.where(c // BLK <= r // BLK, 0.0, NEG).astype(jnp.float32)

        ctx_heads = []
        for h in range(HQ):
            hs = slice(h * DH, (h + 1) * DH)
            qh = q[:, hs]
            s_chunks = []
            for j in range(N_DEV):
                kj = k_all[j, :, hs]
                sj = lax.dot_general(
                    qh, kj, (((1,), (1,)), ((), ())),
                    preferred_element_type=jnp.float32,
                ) * SCALE
                if j == 0:
                    sj = sj + local_bias
                else:
                    sj = sj + jnp.where(my >= j, 0.0, NEG)
                s_chunks.append(sj)
            s = jnp.concatenate(s_chunks, axis=1)
            m = jnp.max(s, axis=1, keepdims=True)
            p = jnp.exp(s - m)
            d = jnp.sum(p, axis=1, keepdims=True)
            w = (p / d).astype(jnp.bfloat16)
            vh = jnp.concatenate(
                [v_all[j, :, hs] for j in range(N_DEV)], axis=0
            )
            ctx_heads.append(
                lax.dot_general(
                    w, vh, (((1,), (0,)), ((), ())),
                    preferred_element_type=jnp.float32,
                ).astype(jnp.bfloat16)
            )

        ctx = jnp.concatenate(ctx_heads, axis=1)
        out_ref[...] = lax.dot_general(
            ctx, wo_ref[...].astype(jnp.bfloat16),
            (((1,), (0,)), ((), ())),
            preferred_element_type=jnp.float32,
        )

    out = pl.pallas_call(
        body,
        out_shape=jax.ShapeDtypeStruct((S_PER, D_MODEL), jnp.float32),
        in_specs=[pl.BlockSpec(memory_space=pltpu.VMEM)] * 5,
        out_specs=pl.BlockSpec(memory_space=pltpu.VMEM),
        scratch_shapes=[
            pltpu.VMEM((N_DEV, S_PER, HQ * DH), jnp.bfloat16),
            pltpu.VMEM((N_DEV, S_PER, HQ * DH), jnp.bfloat16),
            pltpu.SemaphoreType.DMA((N_DEV - 1,)),
            pltpu.SemaphoreType.DMA((N_DEV - 1,)),
            pltpu.SemaphoreType.DMA((N_DEV - 1,)),
            pltpu.SemaphoreType.DMA((N_DEV - 1,)),
        ],
        compiler_params=pltpu.CompilerParams(collective_id=0),
    )(x2, Wq, k2, v2, Wo)
    return out.reshape(1, S_PER, D_MODEL)


# baseline (device time: 237396 ns/iter reference)
import jax
import jax.numpy as jnp
from jax import lax
from jax.experimental import pallas as pl
from jax.experimental.pallas import tpu as pltpu

N_DEV = 4
S_PER = 1024
D_MODEL = 1024
HQ = 8
DH = 128
BLK = 64
SCALE = 0.08838834764831843
NEG = -1e9


def kernel(x, Wq, K_ext, V_ext, Wo):
    x2 = x.reshape(S_PER, D_MODEL)
    k2 = K_ext.reshape(S_PER, HQ * DH)
    v2 = V_ext.reshape(S_PER, HQ * DH)

    def body(x_ref, wq_ref, k_ref, v_ref, wo_ref, out_ref,
             k_all, v_all, ksend, krecv, vsend, vrecv):
        my = lax.axis_index("i")
        left = lax.rem(my + (N_DEV - 1), N_DEV)
        right = lax.rem(my + 1, N_DEV)

        barrier_sem = pltpu.get_barrier_semaphore()
        for nbr in (left, right):
            pl.semaphore_signal(
                barrier_sem, inc=1,
                device_id=(nbr,), device_id_type=pl.DeviceIdType.MESH,
            )
        pl.semaphore_wait(barrier_sem, 2)

        k_all[0] = k_ref[...].astype(jnp.bfloat16)
        v_all[0] = v_ref[...].astype(jnp.bfloat16)

        for h in range(1, N_DEV):
            kr = pltpu.make_async_remote_copy(
                src_ref=k_all.at[h - 1], dst_ref=k_all.at[h],
                send_sem=ksend.at[h - 1], recv_sem=krecv.at[h - 1],
                device_id=(right,), device_id_type=pl.DeviceIdType.MESH,
            )
            vr = pltpu.make_async_remote_copy(
                src_ref=v_all.at[h - 1], dst_ref=v_all.at[h],
                send_sem=vsend.at[h - 1], recv_sem=vrecv.at[h - 1],
                device_id=(right,), device_id_type=pl.DeviceIdType.MESH,
            )
            kr.start()
            vr.start()
            kr.wait()
            vr.wait()

        q = lax.dot_general(
            x_ref[...].astype(jnp.bfloat16),
            wq_ref[...].astype(jnp.bfloat16),
            (((1,), (0,)), ((), ())),
            preferred_element_type=jnp.float32,
        ).astype(jnp.bfloat16)

        r = lax.broadcasted_iota(jnp.int32, (S_PER, S_PER), 0)
        c = lax.broadcasted_iota(jnp.int32, (S_PER, S_PER), 1)
        local_bias = jnp.where(c // BLK <= r // BLK, 0.0, NEG).astype(jnp.float32)

        ctx_heads = []
        for h in range(HQ):
            hs = slice(h * DH, (h + 1) * DH)
            qh = q[:, hs]
            s_chunks = []
            for j in range(N_DEV):
                kj = k_all[j, :, hs]
                sj = lax.dot_general(
                    qh, kj, (((1,), (1,)), ((), ())),
                    preferred_element_type=jnp.float32,
                ) * SCALE
                if j == 0:
                    sj = sj + local_bias
                else:
                    sj = sj + jnp.where(my >= j, 0.0, NEG)
                s_chunks.append(sj)
            s = jnp.concatenate(s_chunks, axis=1)
            m = jnp.max(s, axis=1, keepdims=True)
            p = jnp.exp(s - m)
            d = jnp.sum(p, axis=1, keepdims=True)
            w = (p / d).astype(jnp.bfloat16)
            vh = jnp.concatenate(
                [v_all[j, :, hs] for j in range(N_DEV)], axis=0
            )
            ctx_heads.append(
                lax.dot_general(
                    w, vh, (((1,), (0,)), ((), ())),
                    preferred_element_type=jnp.float32,
                ).astype(jnp.bfloat16)
            )

        ctx = jnp.concatenate(ctx_heads, axis=1)
        out_ref[...] = lax.dot_general(
            ctx, wo_ref[...].astype(jnp.bfloat16),
            (((1,), (0,)), ((), ())),
            preferred_element_type=jnp.float32,
        )

    out = pl.pallas_call(
        body,
        out_shape=jax.ShapeDtypeStruct((S_PER, D_MODEL), jnp.float32),
        in_specs=[pl.BlockSpec(memory_space=pltpu.VMEM)] * 5,
        out_specs=pl.BlockSpec(memory_space=pltpu.VMEM),
        scratch_shapes=[
            pltpu.VMEM((N_DEV, S_PER, HQ * DH), jnp.bfloat16),
            pltpu.VMEM((N_DEV, S_PER, HQ * DH), jnp.bfloat16),
            pltpu.SemaphoreType.DMA((N_DEV - 1,)),
            pltpu.SemaphoreType.DMA((N_DEV - 1,)),
            pltpu.SemaphoreType.DMA((N_DEV - 1,)),
            pltpu.SemaphoreType.DMA((N_DEV - 1,)),
        ],
        compiler_params=pltpu.CompilerParams(
            collective_id=0, vmem_limit_bytes=100 * 1024 * 1024
        ),
    )(x2, Wq, k2, v2, Wo)
    return out.reshape(1, S_PER, D_MODEL)


# device time: 133244 ns/iter; 1.7817x vs baseline; 1.7817x over previous
import jax
import jax.numpy as jnp
from jax import lax
from jax.experimental import pallas as pl
from jax.experimental.pallas import tpu as pltpu

N_DEV = 4
S_PER = 1024
D_MODEL = 1024
HQ = 8
DH = 128
BLK = 64
SCALE = 0.08838834764831843
NEG = -1e9


def kernel(x, Wq, K_ext, V_ext, Wo):
    x2 = x.reshape(S_PER, D_MODEL).astype(jnp.bfloat16)
    wq = Wq.astype(jnp.bfloat16)
    k2 = K_ext.reshape(S_PER, HQ * DH).astype(jnp.bfloat16)
    v2 = V_ext.reshape(S_PER, HQ * DH).astype(jnp.bfloat16)
    wo = Wo.astype(jnp.bfloat16)

    def body(x_ref, wq_ref, k_ref, v_ref, wo_ref, out_ref,
             k_all, v_all, q_buf, bias_buf, acc_buf, ctx_buf,
             kf_send, kf_recv, vf_send, vf_recv,
             kb_send, kb_recv, vb_send, vb_recv):
        my = lax.axis_index("i")
        left = lax.rem(my + (N_DEV - 1), N_DEV)
        right = lax.rem(my + 1, N_DEV)

        barrier_sem = pltpu.get_barrier_semaphore()
        for nbr in (left, right):
            pl.semaphore_signal(
                barrier_sem, inc=1,
                device_id=(nbr,), device_id_type=pl.DeviceIdType.MESH,
            )
        pl.semaphore_wait(barrier_sem, 2)

        def rdma(buf, s_slot, d_slot, send, recv, dev):
            return pltpu.make_async_remote_copy(
                src_ref=buf.at[s_slot], dst_ref=buf.at[d_slot],
                send_sem=send, recv_sem=recv,
                device_id=(dev,), device_id_type=pl.DeviceIdType.MESH,
            )

        f1k = rdma(k_all, 0, 1, kf_send.at[0], kf_recv.at[0], right)
        f1v = rdma(v_all, 0, 1, vf_send.at[0], vf_recv.at[0], right)
        f2k = rdma(k_all, 1, 2, kf_send.at[1], kf_recv.at[1], right)
        f2v = rdma(v_all, 1, 2, vf_send.at[1], vf_recv.at[1], right)
        b1k = rdma(k_all, 0, 3, kb_send, kb_recv, left)
        b1v = rdma(v_all, 0, 3, vb_send, vb_recv, left)

        k_all[0] = k_ref[...]
        v_all[0] = v_ref[...]
        f1k.start()
        f1v.start()
        b1k.start()
        b1v.start()

        q_buf[...] = lax.dot_general(
            x_ref[...], wq_ref[...], (((1,), (0,)), ((), ())),
            preferred_element_type=jnp.float32,
        ).astype(jnp.bfloat16)

        r = lax.broadcasted_iota(jnp.int32, (S_PER, S_PER), 0)
        c = lax.broadcasted_iota(jnp.int32, (S_PER, S_PER), 1)
        bias_buf[...] = jnp.where(c // BLK <= r // BLK, 0.0, NEG).astype(
            jnp.float32
        )

        lsum = [jnp.zeros((S_PER, 1), jnp.float32) for _ in range(HQ)]

        def process_chunk(j, first, use_local_bias, scalar_bias):
            for h in range(HQ):
                hs = slice(h * DH, (h + 1) * DH)
                s = lax.dot_general(
                    q_buf[:, hs], k_all[j, :, hs], (((1,), (1,)), ((), ())),
                    preferred_element_type=jnp.float32,
                ) * SCALE
                s = s + (bias_buf[...] if use_local_bias else scalar_bias)
                p = jnp.exp(s)
                lsum[h] = lsum[h] + jnp.sum(p, axis=1, keepdims=True)
                pv = lax.dot_general(
                    p.astype(jnp.bfloat16), v_all[j, :, hs],
                    (((1,), (0,)), ((), ())),
                    preferred_element_type=jnp.float32,
                )
                acc_buf[:, hs] = pv if first else acc_buf[:, hs] + pv

        process_chunk(0, True, True, 0.0)

        f1k.wait_recv()
        f2k.start()
        f1v.wait_recv()
        f2v.start()
        process_chunk(1, False, False, jnp.where(my >= 1, 0.0, NEG))

        b1k.wait_recv()
        b1v.wait_recv()
        process_chunk(3, False, False, jnp.where(my >= 3, 0.0, NEG))

        f2k.wait_recv()
        f2v.wait_recv()
        process_chunk(2, False, False, jnp.where(my >= 2, 0.0, NEG))

        for h in range(HQ):
            hs = slice(h * DH, (h + 1) * DH)
            ctx_buf[:, hs] = (acc_buf[:, hs] / lsum[h]).astype(jnp.bfloat16)
        out_ref[...] = lax.dot_general(
            ctx_buf[...], wo_ref[...], (((1,), (0,)), ((), ())),
            preferred_element_type=jnp.float32,
        )

        for d in (f1k, f1v, f2k, f2v, b1k, b1v):
            d.wait_send()

    out = pl.pallas_call(
        body,
        out_shape=jax.ShapeDtypeStruct((S_PER, D_MODEL), jnp.float32),
        in_specs=[pl.BlockSpec(memory_space=pltpu.VMEM)] * 5,
        out_specs=pl.BlockSpec(memory_space=pltpu.VMEM),
        scratch_shapes=[
            pltpu.VMEM((N_DEV, S_PER, HQ * DH), jnp.bfloat16),
            pltpu.VMEM((N_DEV, S_PER, HQ * DH), jnp.bfloat16),
            pltpu.VMEM((S_PER, D_MODEL), jnp.bfloat16),
            pltpu.VMEM((S_PER, S_PER), jnp.float32),
            pltpu.VMEM((S_PER, D_MODEL), jnp.float32),
            pltpu.VMEM((S_PER, D_MODEL), jnp.bfloat16),
            pltpu.SemaphoreType.DMA((2,)),
            pltpu.SemaphoreType.DMA((2,)),
            pltpu.SemaphoreType.DMA((2,)),
            pltpu.SemaphoreType.DMA((2,)),
            pltpu.SemaphoreType.DMA,
            pltpu.SemaphoreType.DMA,
            pltpu.SemaphoreType.DMA,
            pltpu.SemaphoreType.DMA,
        ],
        compiler_params=pltpu.CompilerParams(
            collective_id=0, vmem_limit_bytes=100 * 1024 * 1024
        ),
    )(x2, wq, k2, v2, wo)
    return out.reshape(1, S_PER, D_MODEL)


# device time: 125254 ns/iter; 1.8953x vs baseline; 1.0638x over previous
import jax
import jax.numpy as jnp
from jax import lax
from jax.experimental import pallas as pl
from jax.experimental.pallas import tpu as pltpu

N_DEV = 4
S_PER = 1024
D_MODEL = 1024
HQ = 8
DH = 128
BLK = 64
SCALE = 0.08838834764831843
NEG = -1e9


def kernel(x, Wq, K_ext, V_ext, Wo):
    x2 = x.reshape(S_PER, D_MODEL).astype(jnp.bfloat16)
    wq = Wq.astype(jnp.bfloat16)
    k2 = K_ext.reshape(S_PER, HQ * DH).astype(jnp.bfloat16)
    v2 = V_ext.reshape(S_PER, HQ * DH).astype(jnp.bfloat16)
    wo = Wo.astype(jnp.bfloat16)

    def body(x_ref, wq_ref, k_ref, v_ref, wo_ref, out_ref,
             k_all, v_all, q_buf, bias_buf, acc_buf, ctx_buf,
             kf_send, kf_recv, vf_send, vf_recv,
             kb_send, kb_recv, vb_send, vb_recv):
        my = lax.axis_index("i")
        left = lax.rem(my + (N_DEV - 1), N_DEV)
        right = lax.rem(my + 1, N_DEV)

        barrier_sem = pltpu.get_barrier_semaphore()
        for nbr in (left, right):
            pl.semaphore_signal(
                barrier_sem, inc=1,
                device_id=(nbr,), device_id_type=pl.DeviceIdType.MESH,
            )
        pl.semaphore_wait(barrier_sem, 2)

        def head_rdma(buf, s_slot, d_slot, h, send, recv, dev):
            hs = pl.ds(h * DH, DH)
            return pltpu.make_async_remote_copy(
                src_ref=buf.at[s_slot, :, hs], dst_ref=buf.at[d_slot, :, hs],
                send_sem=send, recv_sem=recv,
                device_id=(dev,), device_id_type=pl.DeviceIdType.MESH,
            )

        f1k = [head_rdma(k_all, 0, 1, h, kf_send.at[0, h], kf_recv.at[0, h], right)
               for h in range(HQ)]
        f1v = [head_rdma(v_all, 0, 1, h, vf_send.at[0, h], vf_recv.at[0, h], right)
               for h in range(HQ)]
        f2k = [head_rdma(k_all, 1, 2, h, kf_send.at[1, h], kf_recv.at[1, h], right)
               for h in range(HQ)]
        f2v = [head_rdma(v_all, 1, 2, h, vf_send.at[1, h], vf_recv.at[1, h], right)
               for h in range(HQ)]
        b1k = [head_rdma(k_all, 0, 3, h, kb_send.at[h], kb_recv.at[h], left)
               for h in range(HQ)]
        b1v = [head_rdma(v_all, 0, 3, h, vb_send.at[h], vb_recv.at[h], left)
               for h in range(HQ)]

        k_all[0] = k_ref[...]
        v_all[0] = v_ref[...]
        for h in range(HQ):
            f1k[h].start()
            f1v[h].start()
        for h in range(HQ):
            b1k[h].start()
            b1v[h].start()

        q_buf[...] = lax.dot_general(
            x_ref[...], wq_ref[...], (((1,), (0,)), ((), ())),
            preferred_element_type=jnp.float32,
        ).astype(jnp.bfloat16)

        r = lax.broadcasted_iota(jnp.int32, (S_PER, S_PER), 0)
        c = lax.broadcasted_iota(jnp.int32, (S_PER, S_PER), 1)
        bias_buf[...] = jnp.where(c // BLK <= r // BLK, 0.0, NEG).astype(
            jnp.bfloat16
        )

        lsum = [jnp.zeros((S_PER, 1), jnp.float32) for _ in range(HQ)]

        def process(j, h, first, use_local_bias, scalar_bias):
            hs = slice(h * DH, (h + 1) * DH)
            s = lax.dot_general(
                q_buf[:, hs], k_all[j, :, hs], (((1,), (1,)), ((), ())),
                preferred_element_type=jnp.float32,
            ).astype(jnp.bfloat16) * jnp.bfloat16(SCALE)
            s = s + (bias_buf[...] if use_local_bias else scalar_bias)
            p = jnp.exp(s)
            lsum[h] = lsum[h] + jnp.sum(
                p, axis=1, keepdims=True, dtype=jnp.float32
            )
            pv = lax.dot_general(
                p, v_all[j, :, hs], (((1,), (0,)), ((), ())),
                preferred_element_type=jnp.float32,
            )
            acc_buf[:, hs] = pv if first else acc_buf[:, hs] + pv

        for h in range(HQ):
            process(0, h, True, True, 0.0)

        bias1 = jnp.where(my >= 1, 0.0, NEG).astype(jnp.bfloat16)
        for h in range(HQ):
            f1k[h].wait_recv()
            f2k[h].start()
            f1v[h].wait_recv()
            f2v[h].start()
            process(1, h, False, False, bias1)

        bias3 = jnp.where(my >= 3, 0.0, NEG).astype(jnp.bfloat16)
        for h in range(HQ):
            b1k[h].wait_recv()
            b1v[h].wait_recv()
            process(3, h, False, False, bias3)

        bias2 = jnp.where(my >= 2, 0.0, NEG).astype(jnp.bfloat16)
        for h in range(HQ):
            f2k[h].wait_recv()
            f2v[h].wait_recv()
            process(2, h, False, False, bias2)

        for h in range(HQ):
            hs = slice(h * DH, (h + 1) * DH)
            ctx_buf[:, hs] = (acc_buf[:, hs] / lsum[h]).astype(jnp.bfloat16)
        out_ref[...] = lax.dot_general(
            ctx_buf[...], wo_ref[...], (((1,), (0,)), ((), ())),
            preferred_element_type=jnp.float32,
        )

        for ds in (f1k, f1v, f2k, f2v, b1k, b1v):
            for d in ds:
                d.wait_send()

    out = pl.pallas_call(
        body,
        out_shape=jax.ShapeDtypeStruct((S_PER, D_MODEL), jnp.float32),
        in_specs=[pl.BlockSpec(memory_space=pltpu.VMEM)] * 5,
        out_specs=pl.BlockSpec(memory_space=pltpu.VMEM),
        scratch_shapes=[
            pltpu.VMEM((N_DEV, S_PER, HQ * DH), jnp.bfloat16),
            pltpu.VMEM((N_DEV, S_PER, HQ * DH), jnp.bfloat16),
            pltpu.VMEM((S_PER, D_MODEL), jnp.bfloat16),
            pltpu.VMEM((S_PER, S_PER), jnp.bfloat16),
            pltpu.VMEM((S_PER, D_MODEL), jnp.float32),
            pltpu.VMEM((S_PER, D_MODEL), jnp.bfloat16),
            pltpu.SemaphoreType.DMA((2, HQ)),
            pltpu.SemaphoreType.DMA((2, HQ)),
            pltpu.SemaphoreType.DMA((2, HQ)),
            pltpu.SemaphoreType.DMA((2, HQ)),
            pltpu.SemaphoreType.DMA((HQ,)),
            pltpu.SemaphoreType.DMA((HQ,)),
            pltpu.SemaphoreType.DMA((HQ,)),
            pltpu.SemaphoreType.DMA((HQ,)),
        ],
        compiler_params=pltpu.CompilerParams(
            collective_id=0, vmem_limit_bytes=100 * 1024 * 1024
        ),
    )(x2, wq, k2, v2, wo)
    return out.reshape(1, S_PER, D_MODEL)


# device time: 119063 ns/iter; 1.9939x vs baseline; 1.0520x over previous
import jax
import jax.numpy as jnp
import math
from jax import lax
from jax.experimental import pallas as pl
from jax.experimental.pallas import tpu as pltpu

N_DEV = 4
S_PER = 1024
D_MODEL = 1024
HQ = 8
DH = 128
BLK = 64
SCALE = 0.08838834764831843
Q_SCALE = SCALE * math.log2(math.e)
NEG = -1e9
HALF = S_PER // 2


def kernel(x, Wq, K_ext, V_ext, Wo):
    x2 = x.reshape(S_PER, D_MODEL)
    k2 = K_ext.reshape(S_PER, HQ * DH).astype(jnp.bfloat16)
    v2 = V_ext.reshape(S_PER, HQ * DH).astype(jnp.bfloat16)

    def body(x_ref, wq_ref, k_ref, v_ref, wo_ref, out_ref,
             k_all, v_all, q_buf, bias_buf, acc_buf, ctx_buf, lsum_buf,
             kf_send, kf_recv, vf_send, vf_recv,
             kb_send, kb_recv, vb_send, vb_recv):
        my = lax.axis_index("i")
        left = lax.rem(my + (N_DEV - 1), N_DEV)
        right = lax.rem(my + 1, N_DEV)

        barrier_sem = pltpu.get_barrier_semaphore()
        for nbr in (left, right):
            pl.semaphore_signal(
                barrier_sem, inc=1,
                device_id=(nbr,), device_id_type=pl.DeviceIdType.MESH,
            )
        pl.semaphore_wait(barrier_sem, 2)

        def head_rdma(buf, s_slot, d_slot, h, send, recv, dev):
            hs = pl.ds(h * DH, DH)
            return pltpu.make_async_remote_copy(
                src_ref=buf.at[s_slot, :, hs], dst_ref=buf.at[d_slot, :, hs],
                send_sem=send, recv_sem=recv,
                device_id=(dev,), device_id_type=pl.DeviceIdType.MESH,
            )

        f1k = [head_rdma(k_all, 0, 1, h, kf_send.at[0, h], kf_recv.at[0, h], right)
               for h in range(HQ)]
        f1v = [head_rdma(v_all, 0, 1, h, vf_send.at[0, h], vf_recv.at[0, h], right)
               for h in range(HQ)]
        f2k = [head_rdma(k_all, 1, 2, h, kf_send.at[1, h], kf_recv.at[1, h], right)
               for h in range(HQ)]
        f2v = [head_rdma(v_all, 1, 2, h, vf_send.at[1, h], vf_recv.at[1, h], right)
               for h in range(HQ)]
        b1k = [head_rdma(k_all, 0, 3, h, kb_send.at[h], kb_recv.at[h], left)
               for h in range(HQ)]
        b1v = [head_rdma(v_all, 0, 3, h, vb_send.at[h], vb_recv.at[h], left)
               for h in range(HQ)]

        k_all[0] = k_ref[...]
        v_all[0] = v_ref[...]
        for h in range(HQ):
            f1k[h].start()
            f1v[h].start()
        for h in range(HQ):
            b1k[h].start()
            b1v[h].start()

        q_buf[...] = (
            lax.dot_general(
                x_ref[...].astype(jnp.bfloat16),
                wq_ref[...].astype(jnp.bfloat16),
                (((1,), (0,)), ((), ())),
                preferred_element_type=jnp.float32,
            ) * Q_SCALE
        ).astype(jnp.bfloat16)

        r = lax.broadcasted_iota(jnp.int32, (S_PER, S_PER), 0)
        c = lax.broadcasted_iota(jnp.int32, (S_PER, S_PER), 1)
        bias_buf[...] = jnp.where(c // BLK <= r // BLK, 0.0, NEG).astype(
            jnp.bfloat16
        )

        def qk_exp(rows, hs, j, cols):
            s = lax.dot_general(
                q_buf[rows, hs], k_all[j, cols, hs], (((1,), (1,)), ((), ())),
                preferred_element_type=jnp.float32,
            ).astype(jnp.bfloat16)
            return s

        def process(j, h):
            hs = slice(h * DH, (h + 1) * DH)
            p = jnp.exp2(qk_exp(slice(None), hs, j, slice(None)))
            lsum_buf[h] = lsum_buf[h] + jnp.sum(
                p, axis=1, keepdims=True, dtype=jnp.float32
            )
            acc_buf[:, hs] = acc_buf[:, hs] + lax.dot_general(
                p, v_all[j, :, hs], (((1,), (0,)), ((), ())),
                preferred_element_type=jnp.float32,
            )

        lo = slice(0, HALF)
        for h in range(HQ):
            hs = slice(h * DH, (h + 1) * DH)
            pA = jnp.exp2(qk_exp(lo, hs, 0, lo) + bias_buf[lo, lo])
            lsum_buf[h, lo] = jnp.sum(pA, axis=1, keepdims=True,
                                      dtype=jnp.float32)
            acc_buf[lo, hs] = lax.dot_general(
                pA, v_all[0, lo, hs], (((1,), (0,)), ((), ())),
                preferred_element_type=jnp.float32,
            )
            hi = slice(HALF, S_PER)
            pB = jnp.exp2(qk_exp(hi, hs, 0, slice(None)) + bias_buf[hi, :])
            lsum_buf[h, hi] = jnp.sum(pB, axis=1, keepdims=True,
                                      dtype=jnp.float32)
            acc_buf[hi, hs] = lax.dot_general(
                pB, v_all[0, :, hs], (((1,), (0,)), ((), ())),
                preferred_element_type=jnp.float32,
            )

        for h in range(HQ):
            f1k[h].wait_recv()
            f2k[h].start()
            f1v[h].wait_recv()
            f2v[h].start()

            @pl.when(my >= 1)
            def _(h=h):
                process(1, h)

        for h in range(HQ):
            b1k[h].wait_recv()
            b1v[h].wait_recv()

            @pl.when(my >= 3)
            def _(h=h):
                process(3, h)

        for h in range(HQ):
            f2k[h].wait_recv()
            f2v[h].wait_recv()

            @pl.when(my >= 2)
            def _(h=h):
                process(2, h)

        for h in range(HQ):
            hs = slice(h * DH, (h + 1) * DH)
            ctx_buf[:, hs] = (
                acc_buf[:, hs] * (1.0 / lsum_buf[h])
            ).astype(jnp.bfloat16)
        out_ref[...] = lax.dot_general(
            ctx_buf[...], wo_ref[...].astype(jnp.bfloat16),
            (((1,), (0,)), ((), ())),
            preferred_element_type=jnp.float32,
        )

        for ds in (f1k, f1v, f2k, f2v, b1k, b1v):
            for d in ds:
                d.wait_send()

    out = pl.pallas_call(
        body,
        out_shape=jax.ShapeDtypeStruct((S_PER, D_MODEL), jnp.float32),
        in_specs=[pl.BlockSpec(memory_space=pltpu.VMEM)] * 5,
        out_specs=pl.BlockSpec(memory_space=pltpu.VMEM),
        scratch_shapes=[
            pltpu.VMEM((N_DEV, S_PER, HQ * DH), jnp.bfloat16),
            pltpu.VMEM((N_DEV, S_PER, HQ * DH), jnp.bfloat16),
            pltpu.VMEM((S_PER, D_MODEL), jnp.bfloat16),
            pltpu.VMEM((S_PER, S_PER), jnp.bfloat16),
            pltpu.VMEM((S_PER, D_MODEL), jnp.float32),
            pltpu.VMEM((S_PER, D_MODEL), jnp.bfloat16),
            pltpu.VMEM((HQ, S_PER, 1), jnp.float32),
            pltpu.SemaphoreType.DMA((2, HQ)),
            pltpu.SemaphoreType.DMA((2, HQ)),
            pltpu.SemaphoreType.DMA((2, HQ)),
            pltpu.SemaphoreType.DMA((2, HQ)),
            pltpu.SemaphoreType.DMA((HQ,)),
            pltpu.SemaphoreType.DMA((HQ,)),
            pltpu.SemaphoreType.DMA((HQ,)),
            pltpu.SemaphoreType.DMA((HQ,)),
        ],
        compiler_params=pltpu.CompilerParams(
            collective_id=0, vmem_limit_bytes=100 * 1024 * 1024
        ),
    )(x2, Wq, k2, v2, Wo)
    return out.reshape(1, S_PER, D_MODEL)
